# Initial kernel scaffold; baseline (speedup 1.0000x reference)
import jax
import jax.numpy as jnp
from jax.experimental import pallas as pl


def kernel(x, edge_index, W0, b0, cW0, cb0, g0, be0, W1, b1, cW1, cb1, g1, be1, W2, b2, cW2, cb2, g2, be2, Wres, bres):
    raise NotImplementedError("write your pallas kernel here")


# speedup vs baseline: 7.4172x; 7.4172x over previous
import functools

import numpy as np
import jax
import jax.numpy as jnp
from jax import lax
from jax.experimental import pallas as pl
from jax.experimental.pallas import tpu as pltpu
from jax.experimental.pallas import tpu_sc as plsc

D_HID = 32
N_BASIS = 4
N_CLASS = 2
H_PARAM = 0.075

EB = 128
ROW_BLK = 2000


def _avg_cross_basis(d=D_HID, N=N_BASIS, c=N_CLASS, h=H_PARAM):
    basis = np.zeros((c, N, d, d), dtype=np.float64)
    for i in range(c):
        for k in range(N):
            theta = h * (k + 1) * (i + 1)
            cs, sn = np.cos(theta), np.sin(theta)
            Rm = np.eye(d)
            for b in range(0, d - 1, 2):
                Rm[b, b] = cs
                Rm[b, b + 1] = -sn
                Rm[b + 1, b] = sn
                Rm[b + 1, b + 1] = cs
            basis[i, k] = Rm
    inv = np.linalg.inv(basis)
    bm = basis / np.linalg.eigvals(basis.reshape(-1, d, d)).real.max()
    inv = inv / np.linalg.eigvals(inv.reshape(-1, d, d)).real.max()
    combine = np.empty((c, c, N, d, d))
    for i in range(c):
        for j in range(c):
            if i == j:
                combine[i, j] = np.broadcast_to(np.eye(d), (N, d, d))
            else:
                combine[i, j] = np.einsum('bij,bjk->bik', bm[i], inv[j])
    M = 2 ** (N // 2)
    cross = np.empty((c, c, M, d, d))
    for i in range(c):
        for j in range(c):
            if i == j:
                cross[i, j] = np.broadcast_to(np.eye(d), (M, d, d))
            else:
                for idx in range(M):
                    seq = [(idx >> k) & 1 for k in range(N // 2)]
                    if i < j:
                        mats = [combine[i, j][2 * k + bit] for k, bit in enumerate(seq)]
                    else:
                        seq = seq[::-1]
                        mats = [combine[i, j][2 * (N // 2 - k) - bit - 1] for k, bit in enumerate(seq)]
                    r = mats[0]
                    for m in mats[1:]:
                        r = r @ m
                    cross[i, j, idx] = r
    return cross.mean(axis=2).astype(np.float32)


_M_AVG = _avg_cross_basis()
_M01T = np.ascontiguousarray(_M_AVG[0, 1].T)
_M10T = np.ascontiguousarray(_M_AVG[1, 0].T)


def _class_tables(hb, cWt, cb, m01t, m10t):
    lg = jnp.dot(hb, cWt, preferred_element_type=jnp.float32) + cb
    m = jnp.max(lg, axis=1, keepdims=True)
    e = jnp.exp(lg - m)
    p = e / jnp.sum(e, axis=1, keepdims=True)
    t1 = jnp.dot(hb, m01t, preferred_element_type=jnp.float32)
    t2 = jnp.dot(hb, m10t, preferred_element_type=jnp.float32)
    Ab = p[:, 0:1] * hb + p[:, 1:2] * t1
    Bb = p[:, 1:2] * hb + p[:, 0:1] * t2
    return p, Ab, Bb


def _head0_body(x_ref, Wt_ref, b_ref, cWt_ref, cb_ref, m01_ref, m10_ref,
                h_ref, p_ref, a_ref, b2_ref):
    hb = jnp.dot(x_ref[...], Wt_ref[...], preferred_element_type=jnp.float32) + b_ref[...]
    p, Ab, Bb = _class_tables(hb, cWt_ref[...], cb_ref[...], m01_ref[...], m10_ref[...])
    h_ref[...] = hb
    p_ref[...] = p
    a_ref[...] = Ab
    b2_ref[...] = Bb


def _headn_body(n_nodes, o_ref, st_ref, g_ref, be_ref, Wt_ref, b_ref,
                cWt_ref, cb_ref, m01_ref, m10_ref, h_ref, p_ref, a_ref, b2_ref):
    mu = st_ref[0:1, :] / n_nodes
    var = st_ref[1:2, :] / n_nodes - mu * mu
    inv = g_ref[...] * lax.rsqrt(var + 1e-5)
    z = jnp.maximum((o_ref[...] - mu) * inv + be_ref[...], 0.0)
    hb = jnp.dot(z, Wt_ref[...], preferred_element_type=jnp.float32) + b_ref[...]
    p, Ab, Bb = _class_tables(hb, cWt_ref[...], cb_ref[...], m01_ref[...], m10_ref[...])
    h_ref[...] = hb
    p_ref[...] = p
    a_ref[...] = Ab
    b2_ref[...] = Bb


def _combine_body(h_ref, p_ref, sa_ref, sb_ref, deg_ref, o_ref, st_ref):
    p = p_ref[...]
    agg = p[:, 0:1] * sa_ref[...] + p[:, 1:2] * sb_ref[...]
    o = h_ref[...] + agg / jnp.maximum(deg_ref[...], 1.0)
    o_ref[...] = o

    @pl.when(pl.program_id(0) == 0)
    def _():
        st_ref[...] = jnp.zeros_like(st_ref)

    st_ref[...] += jnp.concatenate(
        [jnp.sum(o, axis=0, keepdims=True), jnp.sum(o * o, axis=0, keepdims=True)], axis=0)


def _final_body(n_nodes, o_ref, st_ref, g_ref, be_ref, Wt_ref, b_ref, out_ref):
    mu = st_ref[0:1, :] / n_nodes
    var = st_ref[1:2, :] / n_nodes - mu * mu
    inv = g_ref[...] * lax.rsqrt(var + 1e-5)
    z = jnp.maximum((o_ref[...] - mu) * inv + be_ref[...], 0.0)
    out_ref[...] = jnp.dot(z, Wt_ref[...], preferred_element_type=jnp.float32) + b_ref[...]


def _row_spec(r, c):
    return pl.BlockSpec((r, c), lambda i: (i, 0))


def _full_spec(shape):
    return pl.BlockSpec(shape, lambda i: tuple(0 for _ in shape))


def _tc_head0(x, Wt, b, cWt, cb, n, rb):
    grid = (n // rb,)
    f32 = jnp.float32
    return pl.pallas_call(
        _head0_body,
        grid=grid,
        in_specs=[_row_spec(rb, x.shape[1]), _full_spec(Wt.shape), _full_spec((1, D_HID)),
                  _full_spec(cWt.shape), _full_spec((1, N_CLASS)),
                  _full_spec((D_HID, D_HID)), _full_spec((D_HID, D_HID))],
        out_specs=[_row_spec(rb, D_HID), _row_spec(rb, N_CLASS),
                   _row_spec(rb, D_HID), _row_spec(rb, D_HID)],
        out_shape=[jax.ShapeDtypeStruct((n, D_HID), f32),
                   jax.ShapeDtypeStruct((n, N_CLASS), f32),
                   jax.ShapeDtypeStruct((n, D_HID), f32),
                   jax.ShapeDtypeStruct((n, D_HID), f32)],
    )(x, Wt, b, cWt, cb, _M01T, _M10T)


def _tc_headn(o, st, g, be, Wt, b, cWt, cb, n, rb):
    grid = (n // rb,)
    f32 = jnp.float32
    return pl.pallas_call(
        functools.partial(_headn_body, float(n)),
        grid=grid,
        in_specs=[_row_spec(rb, D_HID), _full_spec((2, D_HID)), _full_spec((1, D_HID)),
                  _full_spec((1, D_HID)), _full_spec(Wt.shape), _full_spec((1, D_HID)),
                  _full_spec(cWt.shape), _full_spec((1, N_CLASS)),
                  _full_spec((D_HID, D_HID)), _full_spec((D_HID, D_HID))],
        out_specs=[_row_spec(rb, D_HID), _row_spec(rb, N_CLASS),
                   _row_spec(rb, D_HID), _row_spec(rb, D_HID)],
        out_shape=[jax.ShapeDtypeStruct((n, D_HID), f32),
                   jax.ShapeDtypeStruct((n, N_CLASS), f32),
                   jax.ShapeDtypeStruct((n, D_HID), f32),
                   jax.ShapeDtypeStruct((n, D_HID), f32)],
    )(o, st, g, be, Wt, b, cWt, cb, _M01T, _M10T)


def _tc_combine(h, p, sa, sb, deg, n, rb):
    grid = (n // rb,)
    f32 = jnp.float32
    return pl.pallas_call(
        _combine_body,
        grid=grid,
        in_specs=[_row_spec(rb, D_HID), _row_spec(rb, N_CLASS),
                  _row_spec(rb, D_HID), _row_spec(rb, D_HID), _row_spec(rb, 1)],
        out_specs=[_row_spec(rb, D_HID), _full_spec((2, D_HID))],
        out_shape=[jax.ShapeDtypeStruct((n, D_HID), f32),
                   jax.ShapeDtypeStruct((2, D_HID), f32)],
    )(h, p, sa, sb, deg)


def _tc_final(o, st, g, be, Wt, b, n, rb):
    grid = (n // rb,)
    return pl.pallas_call(
        functools.partial(_final_body, float(n)),
        grid=grid,
        in_specs=[_row_spec(rb, D_HID), _full_spec((2, D_HID)), _full_spec((1, D_HID)),
                  _full_spec((1, D_HID)), _full_spec(Wt.shape), _full_spec((1, N_CLASS))],
        out_specs=_row_spec(rb, N_CLASS),
        out_shape=jax.ShapeDtypeStruct((n, N_CLASS), jnp.float32),
    )(o, st, g, be, Wt, b)


@functools.cache
def _make_segsum(n_pad, e_pad, with_deg):
    info = plsc.get_sparse_core_info()
    ns = info.num_subcores
    rt = n_pad // ns
    zr = rt // 16
    nbt = e_pad // (EB * ns)
    f32 = jnp.float32
    mesh = plsc.VectorSubcoreMesh(core_axis_name="c", subcore_axis_name="s")

    out_type = [jax.ShapeDtypeStruct((n_pad, D_HID), f32),
                jax.ShapeDtypeStruct((n_pad, D_HID), f32)]
    scratch = [pltpu.VMEM((EB,), jnp.int32),
               pltpu.VMEM((EB,), jnp.int32),
               pltpu.VMEM((EB, D_HID), f32),
               pltpu.VMEM((zr, D_HID), f32),
               pltpu.VMEM_SHARED((n_pad, D_HID), f32),
               pltpu.SemaphoreType.DMA]
    if with_deg:
        out_type.append(jax.ShapeDtypeStruct((n_pad,), f32))
        scratch += [pltpu.VMEM((EB,), f32),
                    pltpu.VMEM((rt,), f32),
                    pltpu.VMEM_SHARED((n_pad,), f32)]

    def body(src_hbm, dst_hbm, ta_hbm, tb_hbm, sa_hbm, sb_hbm, *rest):
        if with_deg:
            (deg_hbm, src_v, dst_v, rows_v, zero_v, acc, sem,
             ones_v, zd_v, accd) = rest
        else:
            src_v, dst_v, rows_v, zero_v, acc, sem = rest
        cid = lax.axis_index("c")
        tid = lax.axis_index("s")

        z16 = jnp.zeros((16,), f32)

        def zrow(i, carry):
            zero_v[i, pl.ds(0, 16)] = z16
            zero_v[i, pl.ds(16, 16)] = z16
            return carry
        lax.fori_loop(0, zr, zrow, 0)
        for k in range(16):
            pltpu.sync_copy(zero_v, acc.at[pl.ds(tid * rt + k * zr, zr)])

        if with_deg:
            o16 = jnp.ones((16,), f32)
            for k in range(EB // 16):
                ones_v[pl.ds(k * 16, 16)] = o16

            def zdrow(i, carry):
                zd_v[pl.ds(i * 16, 16)] = z16
                return carry
            lax.fori_loop(0, rt // 16, zdrow, 0)

            @pl.when(cid == 0)
            def _():
                pltpu.sync_copy(zd_v, accd.at[pl.ds(tid * rt, rt)])

        plsc.subcore_barrier()

        def ebody(i, carry):
            base = (tid * nbt + i) * EB
            pltpu.sync_copy(src_hbm.at[pl.ds(base, EB)], src_v)
            pltpu.sync_copy(dst_hbm.at[pl.ds(base, EB)], dst_v)

            @pl.when(cid == 0)
            def _():
                pltpu.async_copy(ta_hbm.at[src_v], rows_v, sem).wait()

            @pl.when(cid == 1)
            def _():
                pltpu.async_copy(tb_hbm.at[src_v], rows_v, sem).wait()

            pltpu.sync_copy(rows_v, acc.at[dst_v], add=True)
            if with_deg:
                @pl.when(cid == 0)
                def _():
                    pltpu.sync_copy(ones_v, accd.at[dst_v], add=True)
            return carry
        lax.fori_loop(0, nbt, ebody, 0)

        plsc.subcore_barrier()

        @pl.when(cid == 0)
        def _():
            pltpu.sync_copy(acc.at[pl.ds(tid * rt, rt)], sa_hbm.at[pl.ds(tid * rt, rt)])

        @pl.when(cid == 1)
        def _():
            pltpu.sync_copy(acc.at[pl.ds(tid * rt, rt)], sb_hbm.at[pl.ds(tid * rt, rt)])

        if with_deg:
            @pl.when(cid == 0)
            def _():
                pltpu.sync_copy(accd.at[pl.ds(tid * rt, rt)], deg_hbm.at[pl.ds(tid * rt, rt)])

    return pl.kernel(body, out_type=out_type, mesh=mesh, scratch_types=scratch,
                     compiler_params=pltpu.CompilerParams(use_tc_tiling_on_sc=False))


def kernel(x, edge_index, W0, b0, cW0, cb0, g0, be0, W1, b1, cW1, cb1, g1, be1,
           W2, b2, cW2, cb2, g2, be2, Wres, bres):
    n = x.shape[0]
    e = edge_index.shape[1]
    rb = ROW_BLK if n % ROW_BLK == 0 else n

    n_pad = ((n + 256) // 256) * 256
    e_pad = ((e + 2047) // 2048) * 2048
    src = edge_index[0]
    dst = edge_index[1]
    if e_pad > e:
        src = jnp.concatenate([src, jnp.zeros((e_pad - e,), jnp.int32)])
        dst = jnp.concatenate([dst, jnp.full((e_pad - e,), n, jnp.int32)])

    seg0 = _make_segsum(n_pad, e_pad, True)
    segn = _make_segsum(n_pad, e_pad, False)

    layers = [(W0, b0, cW0, cb0, g0, be0), (W1, b1, cW1, cb1, g1, be1),
              (W2, b2, cW2, cb2, g2, be2)]

    h = p = A = B = None
    deg = None
    o = st = None
    for li, (W, b, cW, cb, g, be) in enumerate(layers):
        if li == 0:
            h, p, A, B = _tc_head0(x, W.T, b[None, :], cW.T, cb[None, :], n, rb)
            sa_f, sb_f, deg_f = seg0(src, dst, A, B)
            deg = deg_f[:n, None]
        else:
            h, p, A, B = _tc_headn(o, st, g_prev[None, :], be_prev[None, :],
                                   W.T, b[None, :], cW.T, cb[None, :], n, rb)
            sa_f, sb_f = segn(src, dst, A, B)
        o, st = _tc_combine(h, p, sa_f[:n], sb_f[:n], deg, n, rb)
        g_prev, be_prev = g, be

    return _tc_final(o, st, g_prev[None, :], be_prev[None, :],
                     Wres.T, bres[None, :], n, rb)


# speedup vs baseline: 11.4779x; 1.5475x over previous
import functools

import numpy as np
import jax
import jax.numpy as jnp
from jax import lax
from jax.experimental import pallas as pl
from jax.experimental.pallas import tpu as pltpu
from jax.experimental.pallas import tpu_sc as plsc

D_HID = 32
N_BASIS = 4
N_CLASS = 2
H_PARAM = 0.075

EB = 128
ROW_BLK = 2000


def _avg_cross_basis(d=D_HID, N=N_BASIS, c=N_CLASS, h=H_PARAM):
    basis = np.zeros((c, N, d, d), dtype=np.float64)
    for i in range(c):
        for k in range(N):
            theta = h * (k + 1) * (i + 1)
            cs, sn = np.cos(theta), np.sin(theta)
            Rm = np.eye(d)
            for b in range(0, d - 1, 2):
                Rm[b, b] = cs
                Rm[b, b + 1] = -sn
                Rm[b + 1, b] = sn
                Rm[b + 1, b + 1] = cs
            basis[i, k] = Rm
    inv = np.linalg.inv(basis)
    bm = basis / np.linalg.eigvals(basis.reshape(-1, d, d)).real.max()
    inv = inv / np.linalg.eigvals(inv.reshape(-1, d, d)).real.max()
    combine = np.empty((c, c, N, d, d))
    for i in range(c):
        for j in range(c):
            if i == j:
                combine[i, j] = np.broadcast_to(np.eye(d), (N, d, d))
            else:
                combine[i, j] = np.einsum('bij,bjk->bik', bm[i], inv[j])
    M = 2 ** (N // 2)
    cross = np.empty((c, c, M, d, d))
    for i in range(c):
        for j in range(c):
            if i == j:
                cross[i, j] = np.broadcast_to(np.eye(d), (M, d, d))
            else:
                for idx in range(M):
                    seq = [(idx >> k) & 1 for k in range(N // 2)]
                    if i < j:
                        mats = [combine[i, j][2 * k + bit] for k, bit in enumerate(seq)]
                    else:
                        seq = seq[::-1]
                        mats = [combine[i, j][2 * (N // 2 - k) - bit - 1] for k, bit in enumerate(seq)]
                    r = mats[0]
                    for m in mats[1:]:
                        r = r @ m
                    cross[i, j, idx] = r
    return cross.mean(axis=2).astype(np.float32)


_M_AVG = _avg_cross_basis()
_M01T = np.ascontiguousarray(_M_AVG[0, 1].T)
_M10T = np.ascontiguousarray(_M_AVG[1, 0].T)


def _class_tables(hb, cWt, cb, m01t, m10t):
    lg = jnp.dot(hb, cWt, preferred_element_type=jnp.float32) + cb
    m = jnp.max(lg, axis=1, keepdims=True)
    e = jnp.exp(lg - m)
    p = e / jnp.sum(e, axis=1, keepdims=True)
    t1 = jnp.dot(hb, m01t, preferred_element_type=jnp.float32)
    t2 = jnp.dot(hb, m10t, preferred_element_type=jnp.float32)
    Ab = p[:, 0:1] * hb + p[:, 1:2] * t1
    Bb = p[:, 1:2] * hb + p[:, 0:1] * t2
    return p, Ab, Bb


def _head0_body(x_ref, Wt_ref, b_ref, cWt_ref, cb_ref, m01_ref, m10_ref,
                h_ref, p_ref, a_ref, b2_ref):
    hb = jnp.dot(x_ref[...], Wt_ref[...], preferred_element_type=jnp.float32) + b_ref[...]
    p, Ab, Bb = _class_tables(hb, cWt_ref[...], cb_ref[...], m01_ref[...], m10_ref[...])
    h_ref[...] = hb
    p_ref[...] = p
    a_ref[...] = Ab
    b2_ref[...] = Bb


def _headn_body(n_nodes, o_ref, st_ref, g_ref, be_ref, Wt_ref, b_ref,
                cWt_ref, cb_ref, m01_ref, m10_ref, h_ref, p_ref, a_ref, b2_ref):
    mu = st_ref[0:1, :] / n_nodes
    var = st_ref[1:2, :] / n_nodes - mu * mu
    inv = g_ref[...] * lax.rsqrt(var + 1e-5)
    z = jnp.maximum((o_ref[...] - mu) * inv + be_ref[...], 0.0)
    hb = jnp.dot(z, Wt_ref[...], preferred_element_type=jnp.float32) + b_ref[...]
    p, Ab, Bb = _class_tables(hb, cWt_ref[...], cb_ref[...], m01_ref[...], m10_ref[...])
    h_ref[...] = hb
    p_ref[...] = p
    a_ref[...] = Ab
    b2_ref[...] = Bb


def _combine_body(h_ref, p_ref, sa_ref, sb_ref, deg_ref, o_ref, st_ref):
    p = p_ref[...]
    agg = p[:, 0:1] * sa_ref[...] + p[:, 1:2] * sb_ref[...]
    o = h_ref[...] + agg / jnp.maximum(deg_ref[...], 1.0)
    o_ref[...] = o

    @pl.when(pl.program_id(0) == 0)
    def _():
        st_ref[...] = jnp.zeros_like(st_ref)

    st_ref[...] += jnp.concatenate(
        [jnp.sum(o, axis=0, keepdims=True), jnp.sum(o * o, axis=0, keepdims=True)], axis=0)


def _final_body(n_nodes, o_ref, st_ref, g_ref, be_ref, Wt_ref, b_ref, out_ref):
    mu = st_ref[0:1, :] / n_nodes
    var = st_ref[1:2, :] / n_nodes - mu * mu
    inv = g_ref[...] * lax.rsqrt(var + 1e-5)
    z = jnp.maximum((o_ref[...] - mu) * inv + be_ref[...], 0.0)
    out_ref[...] = jnp.dot(z, Wt_ref[...], preferred_element_type=jnp.float32) + b_ref[...]


def _row_spec(r, c):
    return pl.BlockSpec((r, c), lambda i: (i, 0))


def _full_spec(shape):
    return pl.BlockSpec(shape, lambda i: tuple(0 for _ in shape))


def _tc_head0(x, Wt, b, cWt, cb, n, rb):
    grid = (n // rb,)
    f32 = jnp.float32
    return pl.pallas_call(
        _head0_body,
        grid=grid,
        in_specs=[_row_spec(rb, x.shape[1]), _full_spec(Wt.shape), _full_spec((1, D_HID)),
                  _full_spec(cWt.shape), _full_spec((1, N_CLASS)),
                  _full_spec((D_HID, D_HID)), _full_spec((D_HID, D_HID))],
        out_specs=[_row_spec(rb, D_HID), _row_spec(rb, N_CLASS),
                   _row_spec(rb, D_HID), _row_spec(rb, D_HID)],
        out_shape=[jax.ShapeDtypeStruct((n, D_HID), f32),
                   jax.ShapeDtypeStruct((n, N_CLASS), f32),
                   jax.ShapeDtypeStruct((n, D_HID), f32),
                   jax.ShapeDtypeStruct((n, D_HID), f32)],
    )(x, Wt, b, cWt, cb, _M01T, _M10T)


def _tc_headn(o, st, g, be, Wt, b, cWt, cb, n, rb):
    grid = (n // rb,)
    f32 = jnp.float32
    return pl.pallas_call(
        functools.partial(_headn_body, float(n)),
        grid=grid,
        in_specs=[_row_spec(rb, D_HID), _full_spec((2, D_HID)), _full_spec((1, D_HID)),
                  _full_spec((1, D_HID)), _full_spec(Wt.shape), _full_spec((1, D_HID)),
                  _full_spec(cWt.shape), _full_spec((1, N_CLASS)),
                  _full_spec((D_HID, D_HID)), _full_spec((D_HID, D_HID))],
        out_specs=[_row_spec(rb, D_HID), _row_spec(rb, N_CLASS),
                   _row_spec(rb, D_HID), _row_spec(rb, D_HID)],
        out_shape=[jax.ShapeDtypeStruct((n, D_HID), f32),
                   jax.ShapeDtypeStruct((n, N_CLASS), f32),
                   jax.ShapeDtypeStruct((n, D_HID), f32),
                   jax.ShapeDtypeStruct((n, D_HID), f32)],
    )(o, st, g, be, Wt, b, cWt, cb, _M01T, _M10T)


def _tc_combine(h, p, sa, sb, deg, n, rb):
    grid = (n // rb,)
    f32 = jnp.float32
    return pl.pallas_call(
        _combine_body,
        grid=grid,
        in_specs=[_row_spec(rb, D_HID), _row_spec(rb, N_CLASS),
                  _row_spec(rb, D_HID), _row_spec(rb, D_HID), _row_spec(rb, 1)],
        out_specs=[_row_spec(rb, D_HID), _full_spec((2, D_HID))],
        out_shape=[jax.ShapeDtypeStruct((n, D_HID), f32),
                   jax.ShapeDtypeStruct((2, D_HID), f32)],
    )(h, p, sa, sb, deg)


def _tc_final(o, st, g, be, Wt, b, n, rb):
    grid = (n // rb,)
    return pl.pallas_call(
        functools.partial(_final_body, float(n)),
        grid=grid,
        in_specs=[_row_spec(rb, D_HID), _full_spec((2, D_HID)), _full_spec((1, D_HID)),
                  _full_spec((1, D_HID)), _full_spec(Wt.shape), _full_spec((1, N_CLASS))],
        out_specs=_row_spec(rb, N_CLASS),
        out_shape=jax.ShapeDtypeStruct((n, N_CLASS), jnp.float32),
    )(o, st, g, be, Wt, b)


CH = 2
CE = CH * EB


@functools.cache
def _make_segsum(n_pad, e_pad, with_deg):
    info = plsc.get_sparse_core_info()
    ns = info.num_subcores
    rt = n_pad // ns
    zr = rt // 16
    nct = e_pad // (CE * ns)
    nc2 = nct // 2
    f32 = jnp.float32
    mesh = plsc.VectorSubcoreMesh(core_axis_name="c", subcore_axis_name="s")

    out_type = [jax.ShapeDtypeStruct((n_pad, D_HID), f32),
                jax.ShapeDtypeStruct((n_pad, D_HID), f32)]
    scratch = [pltpu.VMEM((CH, EB), jnp.int32),
               pltpu.VMEM((CH, EB), jnp.int32),
               pltpu.VMEM((CH, EB, D_HID), f32),
               pltpu.VMEM((CH, EB), jnp.int32),
               pltpu.VMEM((CH, EB), jnp.int32),
               pltpu.VMEM((CH, EB, D_HID), f32),
               pltpu.VMEM((zr, D_HID), f32),
               pltpu.VMEM_SHARED((n_pad, D_HID), f32),
               pltpu.SemaphoreType.DMA,
               pltpu.SemaphoreType.DMA,
               pltpu.SemaphoreType.DMA,
               pltpu.SemaphoreType.DMA]
    if with_deg:
        out_type.append(jax.ShapeDtypeStruct((n_pad,), f32))
        scratch += [pltpu.VMEM((CH, EB), f32),
                    pltpu.VMEM((rt,), f32),
                    pltpu.VMEM_SHARED((n_pad,), f32)]

    def body(src_hbm, dst_hbm, ta_hbm, tb_hbm, sa_hbm, sb_hbm, *rest):
        if with_deg:
            (deg_hbm, src0, dst0, rows0, src1, dst1, rows1, zero_v, acc,
             g0, g1, s0, s1, ones_v, zd_v, accd) = rest
        else:
            (src0, dst0, rows0, src1, dst1, rows1, zero_v, acc,
             g0, g1, s0, s1) = rest
        cid = lax.axis_index("c")
        tid = lax.axis_index("s")
        sets = ((src0, dst0, rows0, g0, s0), (src1, dst1, rows1, g1, s1))

        z16 = jnp.zeros((16,), f32)

        def zrow(i, carry):
            zero_v[i, pl.ds(0, 16)] = z16
            zero_v[i, pl.ds(16, 16)] = z16
            return carry
        lax.fori_loop(0, zr, zrow, 0)
        for k in range(16):
            pltpu.sync_copy(zero_v, acc.at[pl.ds(tid * rt + k * zr, zr)])

        if with_deg:
            o16 = jnp.ones((16,), f32)

            def orow(i, carry):
                ones_v[i, pl.ds(0, 16)] = o16
                ones_v[i, pl.ds(16, 16)] = o16
                ones_v[i, pl.ds(32, 16)] = o16
                ones_v[i, pl.ds(48, 16)] = o16
                ones_v[i, pl.ds(64, 16)] = o16
                ones_v[i, pl.ds(80, 16)] = o16
                ones_v[i, pl.ds(96, 16)] = o16
                ones_v[i, pl.ds(112, 16)] = o16
                return carry
            lax.fori_loop(0, CH, orow, 0)

            def zdrow(i, carry):
                zd_v[pl.ds(i * 16, 16)] = z16
                return carry
            lax.fori_loop(0, rt // 16, zdrow, 0)

            @pl.when(cid == 0)
            def _():
                pltpu.sync_copy(zd_v, accd.at[pl.ds(tid * rt, rt)])

        def idx_load(c, s):
            src_s, dst_s = sets[s][0], sets[s][1]
            r0 = (tid * nct + c) * CH
            pltpu.sync_copy(src_hbm.at[pl.ds(r0, CH)], src_s)
            pltpu.sync_copy(dst_hbm.at[pl.ds(r0, CH)], dst_s)

        def fire_gathers(s):
            src_s, rows_s, gsem = sets[s][0], sets[s][2], sets[s][3]

            @pl.when(cid == 0)
            def _():
                for b in range(CH):
                    pltpu.async_copy(ta_hbm.at[src_s.at[b]], rows_s.at[b], gsem)

            @pl.when(cid == 1)
            def _():
                for b in range(CH):
                    pltpu.async_copy(tb_hbm.at[src_s.at[b]], rows_s.at[b], gsem)

        def drain_gathers(s):
            rows_s, gsem = sets[s][2], sets[s][3]
            for b in range(CH):
                pltpu.make_async_copy(ta_hbm.at[pl.ds(0, EB)], rows_s.at[b], gsem).wait()

        def fire_scatters(s):
            dst_s, rows_s, ssem = sets[s][1], sets[s][2], sets[s][4]
            for b in range(CH):
                pltpu.async_copy(rows_s.at[b], acc.at[dst_s.at[b]], ssem, add=True)
            if with_deg:
                @pl.when(cid == 0)
                def _():
                    for b in range(CH):
                        pltpu.async_copy(ones_v.at[b], accd.at[dst_s.at[b]], ssem, add=True)

        def drain_scatters(s):
            rows_s, ssem = sets[s][2], sets[s][4]
            for b in range(CH):
                pltpu.make_async_copy(ta_hbm.at[pl.ds(0, EB)], rows_s.at[b], ssem).wait()
            if with_deg:
                @pl.when(cid == 0)
                def _():
                    for b in range(CH):
                        pltpu.make_async_copy(deg_hbm.at[pl.ds(0, EB)], ones_v.at[b], ssem).wait()

        idx_load(0, 0)
        fire_gathers(0)
        plsc.subcore_barrier()

        def lbody(i, carry):
            c0 = 2 * i
            drain_gathers(0)
            fire_scatters(0)

            @pl.when(i > 0)
            def _():
                drain_scatters(1)

            idx_load(c0 + 1, 1)
            fire_gathers(1)
            drain_gathers(1)
            fire_scatters(1)
            drain_scatters(0)

            @pl.when(i < nc2 - 1)
            def _():
                idx_load(c0 + 2, 0)
                fire_gathers(0)
            return carry
        lax.fori_loop(0, nc2, lbody, 0)
        drain_scatters(1)

        plsc.subcore_barrier()

        @pl.when(cid == 0)
        def _():
            pltpu.sync_copy(acc.at[pl.ds(tid * rt, rt)], sa_hbm.at[pl.ds(tid * rt, rt)])

        @pl.when(cid == 1)
        def _():
            pltpu.sync_copy(acc.at[pl.ds(tid * rt, rt)], sb_hbm.at[pl.ds(tid * rt, rt)])

        if with_deg:
            @pl.when(cid == 0)
            def _():
                pltpu.sync_copy(accd.at[pl.ds(tid * rt, rt)], deg_hbm.at[pl.ds(tid * rt, rt)])

    return pl.kernel(body, out_type=out_type, mesh=mesh, scratch_types=scratch,
                     compiler_params=pltpu.CompilerParams(use_tc_tiling_on_sc=False))


def kernel(x, edge_index, W0, b0, cW0, cb0, g0, be0, W1, b1, cW1, cb1, g1, be1,
           W2, b2, cW2, cb2, g2, be2, Wres, bres):
    n = x.shape[0]
    e = edge_index.shape[1]
    rb = ROW_BLK if n % ROW_BLK == 0 else n

    n_pad = ((n + 256) // 256) * 256
    ep_unit = 16 * CE * 2
    e_pad = ((e + ep_unit - 1) // ep_unit) * ep_unit
    src = edge_index[0]
    dst = edge_index[1]
    if e_pad > e:
        src = jnp.concatenate([src, jnp.zeros((e_pad - e,), jnp.int32)])
        dst = jnp.concatenate([dst, jnp.full((e_pad - e,), n, jnp.int32)])
    src = src.reshape(e_pad // EB, EB)
    dst = dst.reshape(e_pad // EB, EB)

    seg0 = _make_segsum(n_pad, e_pad, True)
    segn = _make_segsum(n_pad, e_pad, False)

    layers = [(W0, b0, cW0, cb0, g0, be0), (W1, b1, cW1, cb1, g1, be1),
              (W2, b2, cW2, cb2, g2, be2)]

    h = p = A = B = None
    deg = None
    o = st = None
    for li, (W, b, cW, cb, g, be) in enumerate(layers):
        if li == 0:
            h, p, A, B = _tc_head0(x, W.T, b[None, :], cW.T, cb[None, :], n, rb)
            sa_f, sb_f, deg_f = seg0(src, dst, A, B)
            deg = deg_f[:n, None]
        else:
            h, p, A, B = _tc_headn(o, st, g_prev[None, :], be_prev[None, :],
                                   W.T, b[None, :], cW.T, cb[None, :], n, rb)
            sa_f, sb_f = segn(src, dst, A, B)
        o, st = _tc_combine(h, p, sa_f[:n], sb_f[:n], deg, n, rb)
        g_prev, be_prev = g, be

    return _tc_final(o, st, g_prev[None, :], be_prev[None, :],
                     Wres.T, bres[None, :], n, rb)
